# Initial kernel scaffold; baseline (speedup 1.0000x reference)
import functools

import jax
import jax.numpy as jnp
from jax.experimental import pallas as pl
from jax.experimental.pallas import tpu as pltpu

_INV_T = 10.0


def _lse_kernel(pos_ref, neg_ref, out_ref, m_ref, s_ref, acc_ref, *, nr, nc, n_rows):
    ri = pl.program_id(0)
    ci = pl.program_id(1)

    @pl.when(jnp.logical_and(ri == 0, ci == 0))
    def _():
        acc_ref[0, 0] = 0.0

    p = pos_ref[:, :] * _INV_T

    @pl.when(ci == 0)
    def _():
        m_ref[:, :] = p
        s_ref[:, :] = jnp.ones_like(p)

    blk = neg_ref[:, :] * _INV_T
    bm = jnp.max(blk, axis=1, keepdims=True)
    m_old = m_ref[:, :]
    m_new = jnp.maximum(m_old, bm)
    s_ref[:, :] = s_ref[:, :] * jnp.exp(m_old - m_new) + jnp.sum(
        jnp.exp(blk - m_new), axis=1, keepdims=True
    )
    m_ref[:, :] = m_new

    @pl.when(ci == nc - 1)
    def _():
        lse = m_ref[:, :] + jnp.log(s_ref[:, :])
        acc_ref[0, 0] += jnp.sum(lse - p)

        @pl.when(ri == nr - 1)
        def _():
            out_ref[0, 0] = acc_ref[0, 0] / n_rows


def kernel(pos, neg):
    n, m = neg.shape
    br = min(512, n)
    bc = min(4096, m)
    nr = n // br
    nc = m // bc
    out = pl.pallas_call(
        functools.partial(_lse_kernel, nr=nr, nc=nc, n_rows=n),
        grid=(nr, nc),
        in_specs=[
            pl.BlockSpec((br, 1), lambda ri, ci: (ri, 0)),
            pl.BlockSpec((br, bc), lambda ri, ci: (ri, ci)),
        ],
        out_specs=pl.BlockSpec((1, 1), lambda ri, ci: (0, 0)),
        out_shape=jax.ShapeDtypeStruct((1, 1), jnp.float32),
        scratch_shapes=[
            pltpu.VMEM((br, 1), jnp.float32),
            pltpu.VMEM((br, 1), jnp.float32),
            pltpu.SMEM((1, 1), jnp.float32),
        ],
        compiler_params=pltpu.CompilerParams(
            dimension_semantics=("arbitrary", "arbitrary"),
        ),
    )(pos, neg)
    return out[0, 0]


# speedup vs baseline: 2.6984x; 2.6984x over previous
import functools

import jax
import jax.numpy as jnp
from jax.experimental import pallas as pl
from jax.experimental.pallas import tpu as pltpu

_INV_T = 10.0


def _lse_kernel(pos_ref, neg_ref, out_ref, m_ref, s_ref, acc_ref, *, nr, nc, n_rows):
    ri = pl.program_id(0)
    ci = pl.program_id(1)

    @pl.when(jnp.logical_and(ri == 0, ci == 0))
    def _():
        acc_ref[0, 0] = 0.0

    p = pos_ref[:, :] * _INV_T

    @pl.when(ci == 0)
    def _():
        m_ref[:, :] = p
        s_ref[:, :] = jnp.ones_like(p)

    blk = neg_ref[:, :] * _INV_T
    bm = jnp.max(blk, axis=1, keepdims=True)
    m_old = m_ref[:, :]
    m_new = jnp.maximum(m_old, bm)
    s_ref[:, :] = s_ref[:, :] * jnp.exp(m_old - m_new) + jnp.sum(
        jnp.exp(blk - m_new), axis=1, keepdims=True
    )
    m_ref[:, :] = m_new

    @pl.when(ci == nc - 1)
    def _():
        lse = m_ref[:, :] + jnp.log(s_ref[:, :])
        acc_ref[0, 0] += jnp.sum(lse - p)

        @pl.when(ri == nr - 1)
        def _():
            out_ref[:, :] = jnp.full((1, 1), acc_ref[0, 0] / n_rows, jnp.float32)


def kernel(pos, neg):
    n, m = neg.shape
    br = min(512, n)
    bc = min(4096, m)
    nr = n // br
    nc = m // bc
    out = pl.pallas_call(
        functools.partial(_lse_kernel, nr=nr, nc=nc, n_rows=n),
        grid=(nr, nc),
        in_specs=[
            pl.BlockSpec((br, 1), lambda ri, ci: (ri, 0)),
            pl.BlockSpec((br, bc), lambda ri, ci: (ri, ci)),
        ],
        out_specs=pl.BlockSpec((1, 1), lambda ri, ci: (0, 0)),
        out_shape=jax.ShapeDtypeStruct((1, 1), jnp.float32),
        scratch_shapes=[
            pltpu.VMEM((br, 1), jnp.float32),
            pltpu.VMEM((br, 1), jnp.float32),
            pltpu.SMEM((1, 1), jnp.float32),
        ],
        compiler_params=pltpu.CompilerParams(
            dimension_semantics=("arbitrary", "arbitrary"),
        ),
    )(pos, neg)
    return out[0, 0]
